# Initial kernel scaffold; baseline (speedup 1.0000x reference)
import jax
import jax.numpy as jnp
from jax.experimental import pallas as pl


def kernel(tokens, memories, keys, Wq, Wg, Wv, Wo, gq, gg, gv, go):
    raise NotImplementedError("write your pallas kernel here")


# speedup vs baseline: 6.9647x; 6.9647x over previous
import functools

import jax
import jax.numpy as jnp
from jax import lax
from jax.experimental import pallas as pl
from jax.experimental.pallas import tpu as pltpu
from jax.experimental.pallas import tpu_sc as plsc

DIM = 1024
HEADS = 4
NUM_KEYS = 256
DQK = 512
DV = 512
TOPK = 8
DH = 128
RMS_EPS = 1.1920929e-07
IDW_EPS = 0.001
BLK = 256


def _rms(x, g):
    return x * lax.rsqrt(jnp.mean(x * x, axis=-1, keepdims=True) + RMS_EPS) * g


def _top8(d):
    iota = lax.broadcasted_iota(jnp.int32, d.shape, 1)
    work = d
    vals, poss = [], []
    for _ in range(TOPK):
        m = jnp.min(work, axis=-1, keepdims=True)
        pos = jnp.min(jnp.where(work == m, iota, jnp.int32(1 << 30)),
                      axis=-1, keepdims=True)
        work = jnp.where(iota == pos, jnp.float32(jnp.inf), work)
        vals.append(m)
        poss.append(pos)
    return (jnp.concatenate(poss, axis=-1), jnp.concatenate(vals, axis=-1))


def _sel8(arr, sel):
    out = jnp.zeros_like(sel) if arr.dtype == jnp.int32 else jnp.zeros(
        sel.shape, arr.dtype)
    for i in range(TOPK):
        out = jnp.where(sel == i, arr[:, i:i + 1], out)
    return out


def _stage_a(tok, k1t, k2t, wq, wgr, wv, gq, gg, gv,
             fidx_o, wts_o, tvn_o, gate_o):
    x = tok[...]
    xq = _rms(x, gq[...])
    q = jnp.dot(xq, wq[...], preferred_element_type=jnp.float32)
    for h in range(HEADS):
        q1 = q[:, h * DH:(h + 1) * DH]
        q2 = q[:, DQK + h * DH:DQK + (h + 1) * DH]
        kt1 = k1t[h]
        kt2 = k2t[h]
        ks1 = jnp.sum(kt1 * kt1, axis=0, keepdims=True)
        ks2 = jnp.sum(kt2 * kt2, axis=0, keepdims=True)
        c1 = jnp.dot(q1, kt1, preferred_element_type=jnp.float32)
        c2 = jnp.dot(q2, kt2, preferred_element_type=jnp.float32)
        d1 = jnp.sum(q1 * q1, axis=-1, keepdims=True) + ks1 - 2.0 * c1 + IDW_EPS
        d2 = jnp.sum(q2 * q2, axis=-1, keepdims=True) + ks2 - 2.0 * c2 + IDW_EPS
        idx1, s1 = _top8(d1)
        idx2, s2 = _top8(d2)
        prod = (s1[:, :, None] * s2[:, None, :]).reshape(-1, TOPK * TOPK)
        pos, pval = _top8(prod)
        g1 = _sel8(idx1, lax.shift_right_logical(pos, 3))
        g2 = _sel8(idx2, jnp.bitwise_and(pos, jnp.int32(7)))
        w = 1.0 / pval
        w = w / jnp.sum(w, axis=-1, keepdims=True)
        fidx_o[:, h * TOPK:(h + 1) * TOPK] = (
            (g1 * NUM_KEYS + g2) * HEADS + h)
        wts_o[:, h * TOPK:(h + 1) * TOPK] = w

    xg = _rms(x, gg[...])
    gate = jax.nn.sigmoid(jnp.sum(xg * wgr[...], axis=-1, keepdims=True))
    xv = _rms(x, gv[...])
    tv = jnp.dot(xv, wv[...], preferred_element_type=jnp.float32)
    for h in range(HEADS):
        th = tv[:, h * DH:(h + 1) * DH]
        mu = jnp.mean(th, axis=-1, keepdims=True)
        ctr = th - mu
        std = jnp.sqrt(jnp.sum(ctr * ctr, axis=-1, keepdims=True) / (DH - 1))
        tvn_o[:, h * DH:(h + 1) * DH] = ctr / jnp.maximum(std, 1e-10)
    gate_o[...] = jnp.broadcast_to(gate, (gate.shape[0], DH))


def _stage_c(vals, tvn, gate, go, wo, out_o):
    g = gate[:, 0:1]
    o = tvn[...] + g * (vals[...] - tvn[...])
    on = _rms(o, go[...])
    out_o[...] = jnp.dot(on, wo[...], preferred_element_type=jnp.float32)


def _sc_gather_combine(table, idx2d, wflat):
    info = plsc.get_sparse_core_info()
    nw = info.num_cores * info.num_subcores
    n_out = idx2d.shape[0] * idx2d.shape[1] // TOPK
    groups_pw = idx2d.shape[0] // nw
    rows_pg = idx2d.shape[1] // TOPK
    mesh = plsc.VectorSubcoreMesh(core_axis_name="c", subcore_axis_name="s")

    @functools.partial(
        pl.kernel,
        out_type=jax.ShapeDtypeStruct((n_out, DH), jnp.float32),
        mesh=mesh,
        scratch_types=[
            pltpu.VMEM((groups_pw, 128), jnp.int32),
            pltpu.VMEM((groups_pw * 128 + 16,), jnp.float32),
            pltpu.VMEM((128, DH), jnp.float32),
            pltpu.VMEM((rows_pg, DH), jnp.float32),
            pltpu.SemaphoreType.DMA,
        ],
    )
    def k(table_h, idx_h, w_h, out_h, idx_v, w_v, rows_v, out_v, sem):
        wid = lax.axis_index("s") * info.num_cores + lax.axis_index("c")
        pltpu.sync_copy(idx_h.at[pl.ds(wid * groups_pw, groups_pw)], idx_v)
        pltpu.sync_copy(w_h.at[pl.ds(wid * groups_pw * 128, groups_pw * 128)],
                        w_v.at[pl.ds(0, groups_pw * 128)])

        def per_group(g, _):
            pltpu.async_copy(table_h.at[idx_v.at[g]], rows_v, sem).wait()

            def per_row(r, _):
                base = g * 128 + r * TOPK
                wvec = w_v[pl.ds(base, 16)]
                ws = [wvec[kk] for kk in range(TOPK)]
                for j in range(DH // 16):
                    acc = jnp.zeros((16,), jnp.float32)
                    for kk in range(TOPK):
                        acc = acc + ws[kk] * rows_v.at[r * TOPK + kk][
                            pl.ds(j * 16, 16)]
                    out_v.at[r][pl.ds(j * 16, 16)] = acc
                return _

            lax.fori_loop(0, rows_pg, per_row, None)
            pltpu.sync_copy(
                out_v,
                out_h.at[pl.ds(wid * groups_pw * rows_pg + g * rows_pg,
                               rows_pg)])
            return _

        lax.fori_loop(0, groups_pw, per_group, None)

    return k(table, idx2d, wflat)


def _run_stage_a(tokens2d, keys, Wq, Wg, Wv, gq, gg, gv):
    n = tokens2d.shape[0]
    grid = n // BLK
    k1t = jnp.transpose(keys[0], (0, 2, 1))
    k2t = jnp.transpose(keys[1], (0, 2, 1))
    const = lambda shape: pl.BlockSpec(shape, lambda i: (0,) * len(shape))
    return pl.pallas_call(
        _stage_a,
        grid=(grid,),
        in_specs=[
            pl.BlockSpec((BLK, DIM), lambda i: (i, 0)),
            const((HEADS, DH, NUM_KEYS)),
            const((HEADS, DH, NUM_KEYS)),
            const((DIM, DQK * 2)),
            const((1, DIM)),
            const((DIM, DV)),
            const((1, DIM)),
            const((1, DIM)),
            const((1, DIM)),
        ],
        out_specs=[
            pl.BlockSpec((BLK, HEADS * TOPK), lambda i: (i, 0)),
            pl.BlockSpec((BLK, HEADS * TOPK), lambda i: (i, 0)),
            pl.BlockSpec((BLK, DV), lambda i: (i, 0)),
            pl.BlockSpec((BLK, DH), lambda i: (i, 0)),
        ],
        out_shape=[
            jax.ShapeDtypeStruct((n, HEADS * TOPK), jnp.int32),
            jax.ShapeDtypeStruct((n, HEADS * TOPK), jnp.float32),
            jax.ShapeDtypeStruct((n, DV), jnp.float32),
            jax.ShapeDtypeStruct((n, DH), jnp.float32),
        ],
    )(tokens2d, k1t, k2t, Wq, Wg.reshape(1, DIM), Wv,
      gq.reshape(1, DIM), gg.reshape(1, DIM), gv.reshape(1, DIM))


def _run_stage_c(values, tvn, gate, go, Wo):
    n = tvn.shape[0]
    grid = n // BLK
    const = lambda shape: pl.BlockSpec(shape, lambda i: (0,) * len(shape))
    return pl.pallas_call(
        _stage_c,
        grid=(grid,),
        in_specs=[
            pl.BlockSpec((BLK, DV), lambda i: (i, 0)),
            pl.BlockSpec((BLK, DV), lambda i: (i, 0)),
            pl.BlockSpec((BLK, DH), lambda i: (i, 0)),
            const((1, DV)),
            const((DV, DIM)),
        ],
        out_specs=pl.BlockSpec((BLK, DIM), lambda i: (i, 0)),
        out_shape=jax.ShapeDtypeStruct((n, DIM), jnp.float32),
    )(values, tvn, gate, go.reshape(1, DV), Wo)


def kernel(tokens, memories, keys, Wq, Wg, Wv, Wo, gq, gg, gv, go):
    b, n, _ = tokens.shape
    tok2d = tokens.reshape(b * n, DIM)
    fidx, wts, tvn, gate = _run_stage_a(tok2d, keys, Wq, Wg, Wv, gq, gg, gv)
    table = memories.reshape(-1, DH)
    idx2d = fidx.reshape(-1, 128)
    wflat = wts.reshape(-1)
    vals = _sc_gather_combine(table, idx2d, wflat)
    out = _run_stage_c(vals.reshape(b * n, DV), tvn, gate, go, Wo)
    return out.reshape(b, n, DIM)


# speedup vs baseline: 14.8952x; 2.1387x over previous
import functools

import jax
import jax.numpy as jnp
from jax import lax
from jax.experimental import pallas as pl
from jax.experimental.pallas import tpu as pltpu
from jax.experimental.pallas import tpu_sc as plsc

DIM = 1024
HEADS = 4
NUM_KEYS = 256
DQK = 512
DV = 512
TOPK = 8
DH = 128
RMS_EPS = 1.1920929e-07
IDW_EPS = 0.001
BLK = 256
TCH = 128


def _rms(x, g):
    return x * lax.rsqrt(jnp.mean(x * x, axis=-1, keepdims=True) + RMS_EPS) * g


def _top8(d):
    iota = lax.broadcasted_iota(jnp.int32, d.shape, 0).astype(jnp.float32)
    work = d
    vals, poss = [], []
    for _ in range(TOPK):
        m = jnp.min(work, axis=0, keepdims=True)
        pos = jnp.min(jnp.where(work == m, iota, jnp.float32(512.0)),
                      axis=0, keepdims=True)
        work = jnp.where(iota == pos, jnp.float32(jnp.inf), work)
        vals.append(m)
        poss.append(pos)
    return (jnp.concatenate(poss, axis=0).astype(jnp.int32),
            jnp.concatenate(vals, axis=0))


def _sel8(arr, sel):
    out = jnp.zeros(sel.shape, arr.dtype)
    for i in range(TOPK):
        out = jnp.where(sel == i, arr[i:i + 1, :], out)
    return out


def _stage_a(tok, k1, k2, wq, wgr, wv, gq, gg, gv,
             fidx_o, wts_o, tvn_o, gate_o):
    x = tok[...]
    xq = _rms(x, gq[...])
    q = jnp.dot(xq, wq[...], preferred_element_type=jnp.float32)
    ones_row = jnp.ones((1, DH), jnp.float32)
    dn_t = (((1,), (1,)), ((), ()))
    for h in range(HEADS):
        k1h = k1[h]
        k2h = k2[h]
        ks1 = jnp.sum(k1h * k1h, axis=-1, keepdims=True)
        ks2 = jnp.sum(k2h * k2h, axis=-1, keepdims=True)
        for c in range(0, q.shape[0], TCH):
            sl = slice(c, c + TCH)
            q1 = q[sl, h * DH:(h + 1) * DH]
            q2 = q[sl, DQK + h * DH:DQK + (h + 1) * DH]
            c1 = lax.dot_general(k1h, q1, dn_t,
                                 preferred_element_type=jnp.float32)
            c2 = lax.dot_general(k2h, q2, dn_t,
                                 preferred_element_type=jnp.float32)
            qs1 = lax.dot_general(ones_row, q1 * q1, dn_t,
                                  preferred_element_type=jnp.float32)
            qs2 = lax.dot_general(ones_row, q2 * q2, dn_t,
                                  preferred_element_type=jnp.float32)
            idx1, v1 = _top8(ks1 - 2.0 * c1)
            idx2, v2 = _top8(ks2 - 2.0 * c2)
            s1 = v1 + (qs1 + IDW_EPS)
            s2 = v2 + (qs2 + IDW_EPS)
            prod = (s1[:, None, :] * s2[None, :, :]).reshape(
                TOPK * TOPK, -1)
            pos, pval = _top8(prod)
            g1 = _sel8(idx1, lax.shift_right_logical(pos, 3))
            g2 = _sel8(idx2, jnp.bitwise_and(pos, jnp.int32(7)))
            w = 1.0 / pval
            w = w / jnp.sum(w, axis=0, keepdims=True)
            fidx_o[sl, h * TOPK:(h + 1) * TOPK] = jnp.transpose(
                (g1 * NUM_KEYS + g2) * HEADS + h)
            wts_o[sl, h * TOPK:(h + 1) * TOPK] = jnp.transpose(w)

    xg = _rms(x, gg[...])
    gate = jax.nn.sigmoid(jnp.sum(xg * wgr[...], axis=-1, keepdims=True))
    xv = _rms(x, gv[...])
    tv = jnp.dot(xv, wv[...], preferred_element_type=jnp.float32)
    for h in range(HEADS):
        th = tv[:, h * DH:(h + 1) * DH]
        mu = jnp.mean(th, axis=-1, keepdims=True)
        ctr = th - mu
        std = jnp.sqrt(jnp.sum(ctr * ctr, axis=-1, keepdims=True) / (DH - 1))
        tvn_o[:, h * DH:(h + 1) * DH] = ctr / jnp.maximum(std, 1e-10)
    gate_o[...] = jnp.broadcast_to(gate, (gate.shape[0], DH))


def _stage_c(vals, tvn, gate, go, wo, out_o):
    g = gate[:, 0:1]
    o = tvn[...] + g * (vals[...] - tvn[...])
    on = _rms(o, go[...])
    out_o[...] = jnp.dot(on, wo[...], preferred_element_type=jnp.float32)


def _sc_gather_combine(table, idx2d, wflat):
    info = plsc.get_sparse_core_info()
    nw = info.num_cores * info.num_subcores
    n_out = idx2d.shape[0] * idx2d.shape[1] // TOPK
    groups_pw = idx2d.shape[0] // nw
    rows_pg = idx2d.shape[1] // TOPK
    mesh = plsc.VectorSubcoreMesh(core_axis_name="c", subcore_axis_name="s")

    @functools.partial(
        pl.kernel,
        out_type=jax.ShapeDtypeStruct((n_out, DH), jnp.float32),
        mesh=mesh,
        scratch_types=[
            pltpu.VMEM((groups_pw, 128), jnp.int32),
            pltpu.VMEM((groups_pw * 128 + 16,), jnp.float32),
            pltpu.VMEM((128, DH), jnp.float32),
            pltpu.VMEM((rows_pg, DH), jnp.float32),
            pltpu.SemaphoreType.DMA,
        ],
    )
    def k(table_h, idx_h, w_h, out_h, idx_v, w_v, rows_v, out_v, sem):
        wid = lax.axis_index("s") * info.num_cores + lax.axis_index("c")
        pltpu.sync_copy(idx_h.at[pl.ds(wid * groups_pw, groups_pw)], idx_v)
        pltpu.sync_copy(w_h.at[pl.ds(wid * groups_pw * 128, groups_pw * 128)],
                        w_v.at[pl.ds(0, groups_pw * 128)])

        def per_group(g, _):
            pltpu.async_copy(table_h.at[idx_v.at[g]], rows_v, sem).wait()

            def per_row(r, _):
                base = g * 128 + r * TOPK
                wvec = w_v[pl.ds(base, 16)]
                ws = [wvec[kk] for kk in range(TOPK)]
                for j in range(DH // 16):
                    acc = jnp.zeros((16,), jnp.float32)
                    for kk in range(TOPK):
                        acc = acc + ws[kk] * rows_v.at[r * TOPK + kk][
                            pl.ds(j * 16, 16)]
                    out_v.at[r][pl.ds(j * 16, 16)] = acc
                return _

            lax.fori_loop(0, rows_pg, per_row, None)
            pltpu.sync_copy(
                out_v,
                out_h.at[pl.ds(wid * groups_pw * rows_pg + g * rows_pg,
                               rows_pg)])
            return _

        lax.fori_loop(0, groups_pw, per_group, None)

    return k(table, idx2d, wflat)


def _run_stage_a(tokens2d, keys, Wq, Wg, Wv, gq, gg, gv):
    n = tokens2d.shape[0]
    grid = n // BLK
    const = lambda shape: pl.BlockSpec(shape, lambda i: (0,) * len(shape))
    return pl.pallas_call(
        _stage_a,
        grid=(grid,),
        in_specs=[
            pl.BlockSpec((BLK, DIM), lambda i: (i, 0)),
            const((HEADS, NUM_KEYS, DH)),
            const((HEADS, NUM_KEYS, DH)),
            const((DIM, DQK * 2)),
            const((1, DIM)),
            const((DIM, DV)),
            const((1, DIM)),
            const((1, DIM)),
            const((1, DIM)),
        ],
        out_specs=[
            pl.BlockSpec((BLK, HEADS * TOPK), lambda i: (i, 0)),
            pl.BlockSpec((BLK, HEADS * TOPK), lambda i: (i, 0)),
            pl.BlockSpec((BLK, DV), lambda i: (i, 0)),
            pl.BlockSpec((BLK, DH), lambda i: (i, 0)),
        ],
        out_shape=[
            jax.ShapeDtypeStruct((n, HEADS * TOPK), jnp.int32),
            jax.ShapeDtypeStruct((n, HEADS * TOPK), jnp.float32),
            jax.ShapeDtypeStruct((n, DV), jnp.float32),
            jax.ShapeDtypeStruct((n, DH), jnp.float32),
        ],
    )(tokens2d, keys[0], keys[1], Wq, Wg.reshape(1, DIM), Wv,
      gq.reshape(1, DIM), gg.reshape(1, DIM), gv.reshape(1, DIM))


def _run_stage_c(values, tvn, gate, go, Wo):
    n = tvn.shape[0]
    grid = n // BLK
    const = lambda shape: pl.BlockSpec(shape, lambda i: (0,) * len(shape))
    return pl.pallas_call(
        _stage_c,
        grid=(grid,),
        in_specs=[
            pl.BlockSpec((BLK, DV), lambda i: (i, 0)),
            pl.BlockSpec((BLK, DV), lambda i: (i, 0)),
            pl.BlockSpec((BLK, DH), lambda i: (i, 0)),
            const((1, DV)),
            const((DV, DIM)),
        ],
        out_specs=pl.BlockSpec((BLK, DIM), lambda i: (i, 0)),
        out_shape=jax.ShapeDtypeStruct((n, DIM), jnp.float32),
    )(values, tvn, gate, go.reshape(1, DV), Wo)


def kernel(tokens, memories, keys, Wq, Wg, Wv, Wo, gq, gg, gv, go):
    b, n, _ = tokens.shape
    tok2d = tokens.reshape(b * n, DIM)
    fidx, wts, tvn, gate = _run_stage_a(tok2d, keys, Wq, Wg, Wv, gq, gg, gv)
    table = memories.reshape(-1, DH)
    idx2d = fidx.reshape(-1, 128)
    wflat = wts.reshape(-1)
    vals = _sc_gather_combine(table, idx2d, wflat)
    out = _run_stage_c(vals.reshape(b * n, DV), tvn, gate, go, Wo)
    return out.reshape(b, n, DIM)


# speedup vs baseline: 15.3495x; 1.0305x over previous
import functools

import jax
import jax.numpy as jnp
from jax import lax
from jax.experimental import pallas as pl
from jax.experimental.pallas import tpu as pltpu
from jax.experimental.pallas import tpu_sc as plsc

DIM = 1024
HEADS = 4
NUM_KEYS = 256
DQK = 512
DV = 512
TOPK = 8
DH = 128
RMS_EPS = 1.1920929e-07
IDW_EPS = 0.001
BLK = 256
TCH = 128


def _rms(x, g):
    return x / jnp.sqrt(jnp.mean(x * x, axis=-1, keepdims=True) + RMS_EPS) * g


def _top8(d):
    iota = lax.broadcasted_iota(jnp.int32, d.shape, 0).astype(jnp.float32)
    work = d
    vals, poss = [], []
    for _ in range(TOPK):
        m = jnp.min(work, axis=0, keepdims=True)
        pos = jnp.min(jnp.where(work == m, iota, jnp.float32(512.0)),
                      axis=0, keepdims=True)
        work = jnp.where(iota == pos, jnp.float32(jnp.inf), work)
        vals.append(m)
        poss.append(pos)
    return (jnp.concatenate(poss, axis=0).astype(jnp.int32),
            jnp.concatenate(vals, axis=0))


def _sel8(arr, sel):
    out = jnp.zeros(sel.shape, arr.dtype)
    for i in range(TOPK):
        out = jnp.where(sel == i, arr[i:i + 1, :], out)
    return out


def _stage_a(tok, k1, k2, wq, wgr, wv, gq, gg, gv,
             fidx_o, wts_o, tvn_o, gate_o):
    x = tok[...]
    xq = _rms(x, gq[...])
    q = jnp.dot(xq, wq[...], preferred_element_type=jnp.float32)
    dn_t = (((1,), (1,)), ((), ()))
    for h in range(HEADS):
        k1h = k1[h]
        k2h = k2[h]
        ks1 = jnp.sum(k1h * k1h, axis=-1, keepdims=True)
        ks2 = jnp.sum(k2h * k2h, axis=-1, keepdims=True)
        for c in range(0, q.shape[0], TCH):
            sl = slice(c, c + TCH)
            q1 = q[sl, h * DH:(h + 1) * DH]
            q2 = q[sl, DQK + h * DH:DQK + (h + 1) * DH]
            c1 = lax.dot_general(k1h, q1, dn_t,
                                 preferred_element_type=jnp.float32)
            c2 = lax.dot_general(k2h, q2, dn_t,
                                 preferred_element_type=jnp.float32)
            qs1 = jnp.transpose(jnp.sum(q1 * q1, axis=-1, keepdims=True))
            qs2 = jnp.transpose(jnp.sum(q2 * q2, axis=-1, keepdims=True))
            idx1, s1 = _top8((qs1 + ks1) - 2.0 * c1 + IDW_EPS)
            idx2, s2 = _top8((qs2 + ks2) - 2.0 * c2 + IDW_EPS)
            prod = (s1[:, None, :] * s2[None, :, :]).reshape(
                TOPK * TOPK, -1)
            pos, pval = _top8(prod)
            g1 = _sel8(idx1, lax.shift_right_logical(pos, 3))
            g2 = _sel8(idx2, jnp.bitwise_and(pos, jnp.int32(7)))
            w = 1.0 / pval
            w = w / jnp.sum(w, axis=0, keepdims=True)
            fidx_o[sl, h * TOPK:(h + 1) * TOPK] = jnp.transpose(
                (g1 * NUM_KEYS + g2) * HEADS + h)
            wts_o[sl, h * TOPK:(h + 1) * TOPK] = jnp.transpose(w)

    xg = _rms(x, gg[...])
    gate = jax.nn.sigmoid(jnp.sum(xg * wgr[...], axis=-1, keepdims=True))
    xv = _rms(x, gv[...])
    tv = jnp.dot(xv, wv[...], preferred_element_type=jnp.float32)
    for h in range(HEADS):
        th = tv[:, h * DH:(h + 1) * DH]
        mu = jnp.mean(th, axis=-1, keepdims=True)
        ctr = th - mu
        std = jnp.sqrt(jnp.sum(ctr * ctr, axis=-1, keepdims=True) / (DH - 1))
        tvn_o[:, h * DH:(h + 1) * DH] = ctr / jnp.maximum(std, 1e-10)
    gate_o[...] = jnp.broadcast_to(gate, (gate.shape[0], DH))


def _stage_c(vals, tvn, gate, go, wo, out_o):
    g = gate[:, 0:1]
    o = tvn[...] + g * (vals[...] - tvn[...])
    on = _rms(o, go[...])
    out_o[...] = jnp.dot(on, wo[...], preferred_element_type=jnp.float32)


def _sc_gather_combine(table, idx2d, wflat):
    info = plsc.get_sparse_core_info()
    nw = info.num_cores * info.num_subcores
    n_out = idx2d.shape[0] * idx2d.shape[1] // TOPK
    groups_pw = idx2d.shape[0] // nw
    rows_pg = idx2d.shape[1] // TOPK
    mesh = plsc.VectorSubcoreMesh(core_axis_name="c", subcore_axis_name="s")

    @functools.partial(
        pl.kernel,
        out_type=jax.ShapeDtypeStruct((n_out, DH), jnp.float32),
        mesh=mesh,
        scratch_types=[
            pltpu.VMEM((groups_pw, 128), jnp.int32),
            pltpu.VMEM((groups_pw * 128 + 16,), jnp.float32),
            pltpu.VMEM((128, DH), jnp.float32),
            pltpu.VMEM((rows_pg, DH), jnp.float32),
            pltpu.SemaphoreType.DMA,
        ],
    )
    def k(table_h, idx_h, w_h, out_h, idx_v, w_v, rows_v, out_v, sem):
        wid = lax.axis_index("s") * info.num_cores + lax.axis_index("c")
        pltpu.sync_copy(idx_h.at[pl.ds(wid * groups_pw, groups_pw)], idx_v)
        pltpu.sync_copy(w_h.at[pl.ds(wid * groups_pw * 128, groups_pw * 128)],
                        w_v.at[pl.ds(0, groups_pw * 128)])

        def per_group(g, _):
            pltpu.async_copy(table_h.at[idx_v.at[g]], rows_v, sem).wait()

            def per_row(r, _):
                base = g * 128 + r * TOPK
                wvec = w_v[pl.ds(base, 16)]
                ws = [wvec[kk] for kk in range(TOPK)]
                for j in range(DH // 16):
                    acc = jnp.zeros((16,), jnp.float32)
                    for kk in range(TOPK):
                        acc = acc + ws[kk] * rows_v.at[r * TOPK + kk][
                            pl.ds(j * 16, 16)]
                    out_v.at[r][pl.ds(j * 16, 16)] = acc
                return _

            lax.fori_loop(0, rows_pg, per_row, None)
            pltpu.sync_copy(
                out_v,
                out_h.at[pl.ds(wid * groups_pw * rows_pg + g * rows_pg,
                               rows_pg)])
            return _

        lax.fori_loop(0, groups_pw, per_group, None)

    return k(table, idx2d, wflat)


def _run_stage_a(tokens2d, keys, Wq, Wg, Wv, gq, gg, gv):
    n = tokens2d.shape[0]
    grid = n // BLK
    const = lambda shape: pl.BlockSpec(shape, lambda i: (0,) * len(shape))
    return pl.pallas_call(
        _stage_a,
        grid=(grid,),
        in_specs=[
            pl.BlockSpec((BLK, DIM), lambda i: (i, 0)),
            const((HEADS, NUM_KEYS, DH)),
            const((HEADS, NUM_KEYS, DH)),
            const((DIM, DQK * 2)),
            const((1, DIM)),
            const((DIM, DV)),
            const((1, DIM)),
            const((1, DIM)),
            const((1, DIM)),
        ],
        out_specs=[
            pl.BlockSpec((BLK, HEADS * TOPK), lambda i: (i, 0)),
            pl.BlockSpec((BLK, HEADS * TOPK), lambda i: (i, 0)),
            pl.BlockSpec((BLK, DV), lambda i: (i, 0)),
            pl.BlockSpec((BLK, DH), lambda i: (i, 0)),
        ],
        out_shape=[
            jax.ShapeDtypeStruct((n, HEADS * TOPK), jnp.int32),
            jax.ShapeDtypeStruct((n, HEADS * TOPK), jnp.float32),
            jax.ShapeDtypeStruct((n, DV), jnp.float32),
            jax.ShapeDtypeStruct((n, DH), jnp.float32),
        ],
    )(tokens2d, keys[0], keys[1], Wq, Wg.reshape(1, DIM), Wv,
      gq.reshape(1, DIM), gg.reshape(1, DIM), gv.reshape(1, DIM))


def _run_stage_c(values, tvn, gate, go, Wo):
    n = tvn.shape[0]
    grid = n // BLK
    const = lambda shape: pl.BlockSpec(shape, lambda i: (0,) * len(shape))
    return pl.pallas_call(
        _stage_c,
        grid=(grid,),
        in_specs=[
            pl.BlockSpec((BLK, DV), lambda i: (i, 0)),
            pl.BlockSpec((BLK, DV), lambda i: (i, 0)),
            pl.BlockSpec((BLK, DH), lambda i: (i, 0)),
            const((1, DV)),
            const((DV, DIM)),
        ],
        out_specs=pl.BlockSpec((BLK, DIM), lambda i: (i, 0)),
        out_shape=jax.ShapeDtypeStruct((n, DIM), jnp.float32),
    )(values, tvn, gate, go.reshape(1, DV), Wo)


def kernel(tokens, memories, keys, Wq, Wg, Wv, Wo, gq, gg, gv, go):
    b, n, _ = tokens.shape
    tok2d = tokens.reshape(b * n, DIM)
    fidx, wts, tvn, gate = _run_stage_a(tok2d, keys, Wq, Wg, Wv, gq, gg, gv)
    table = memories.reshape(-1, DH)
    idx2d = fidx.reshape(-1, 128)
    wflat = wts.reshape(-1)
    vals = _sc_gather_combine(table, idx2d, wflat)
    out = _run_stage_c(vals.reshape(b * n, DV), tvn, gate, go, Wo)
    return out.reshape(b, n, DIM)


# speedup vs baseline: 17.8795x; 1.1648x over previous
import functools

import jax
import jax.numpy as jnp
from jax import lax
from jax.experimental import pallas as pl
from jax.experimental.pallas import tpu as pltpu
from jax.experimental.pallas import tpu_sc as plsc

DIM = 1024
HEADS = 4
NUM_KEYS = 256
DQK = 512
DV = 512
TOPK = 8
DH = 128
RMS_EPS = 1.1920929e-07
IDW_EPS = 0.001
BLK = 256
TCH = 128


def _rms(x, g):
    return x / jnp.sqrt(jnp.mean(x * x, axis=-1, keepdims=True) + RMS_EPS) * g


def _top8(d):
    iota = lax.broadcasted_iota(jnp.int32, d.shape, 0).astype(jnp.float32)
    work = d
    vals, poss = [], []
    for _ in range(TOPK):
        m = jnp.min(work, axis=0, keepdims=True)
        pos = jnp.min(jnp.where(work == m, iota, jnp.float32(512.0)),
                      axis=0, keepdims=True)
        work = jnp.where(iota == pos, jnp.float32(jnp.inf), work)
        vals.append(m)
        poss.append(pos)
    return (jnp.concatenate(poss, axis=0).astype(jnp.int32),
            jnp.concatenate(vals, axis=0))


def _sel8(arr, sel):
    out = jnp.zeros(sel.shape, arr.dtype)
    for i in range(TOPK):
        out = jnp.where(sel == i, arr[i:i + 1, :], out)
    return out


def _stage_a(tok, k1, k2, wq, wgr, wv, gq, gg, gv,
             fidx_o, wts_o, tvn_o, gate_o):
    x = tok[...]
    xq = _rms(x, gq[...])
    q = jnp.dot(xq, wq[...], preferred_element_type=jnp.float32)
    dn_t = (((1,), (1,)), ((), ()))
    for h in range(HEADS):
        k1h = k1[h]
        k2h = k2[h]
        ks1 = jnp.sum(k1h * k1h, axis=-1, keepdims=True)
        ks2 = jnp.sum(k2h * k2h, axis=-1, keepdims=True)
        for c in range(0, q.shape[0], TCH):
            sl = slice(c, c + TCH)
            q1 = q[sl, h * DH:(h + 1) * DH]
            q2 = q[sl, DQK + h * DH:DQK + (h + 1) * DH]
            c1 = lax.dot_general(k1h, q1, dn_t,
                                 preferred_element_type=jnp.float32)
            c2 = lax.dot_general(k2h, q2, dn_t,
                                 preferred_element_type=jnp.float32)
            qs1 = jnp.transpose(jnp.sum(q1 * q1, axis=-1, keepdims=True))
            qs2 = jnp.transpose(jnp.sum(q2 * q2, axis=-1, keepdims=True))
            idx1, s1 = _top8((qs1 + ks1) - 2.0 * c1 + IDW_EPS)
            idx2, s2 = _top8((qs2 + ks2) - 2.0 * c2 + IDW_EPS)
            prod = (s1[:, None, :] * s2[None, :, :]).reshape(
                TOPK * TOPK, -1)
            pos, pval = _top8(prod)
            g1 = _sel8(idx1, lax.shift_right_logical(pos, 3))
            g2 = _sel8(idx2, jnp.bitwise_and(pos, jnp.int32(7)))
            w = 1.0 / pval
            w = w / jnp.sum(w, axis=0, keepdims=True)
            fidx_o[sl, h * TOPK:(h + 1) * TOPK] = jnp.transpose(
                (g1 * NUM_KEYS + g2) * HEADS + h)
            wts_o[sl, h * TOPK:(h + 1) * TOPK] = jnp.transpose(w)

    xg = _rms(x, gg[...])
    gate = jax.nn.sigmoid(jnp.sum(xg * wgr[...], axis=-1, keepdims=True))
    xv = _rms(x, gv[...])
    tv = jnp.dot(xv, wv[...], preferred_element_type=jnp.float32)
    for h in range(HEADS):
        th = tv[:, h * DH:(h + 1) * DH]
        mu = jnp.mean(th, axis=-1, keepdims=True)
        ctr = th - mu
        std = jnp.sqrt(jnp.sum(ctr * ctr, axis=-1, keepdims=True) / (DH - 1))
        tvn_o[:, h * DH:(h + 1) * DH] = ctr / jnp.maximum(std, 1e-10)
    gate_o[...] = jnp.broadcast_to(gate, (gate.shape[0], DH))


def _stage_c(vals, tvn, gate, go, wo, out_o):
    g = gate[:, 0:1]
    o = tvn[...] + g * (vals[...] - tvn[...])
    on = _rms(o, go[...])
    out_o[...] = jnp.dot(on, wo[...], preferred_element_type=jnp.float32)


def _sc_gather_combine(table, idx2d, wflat):
    info = plsc.get_sparse_core_info()
    nw = info.num_cores * info.num_subcores
    n_out = idx2d.shape[0] * idx2d.shape[1] // TOPK
    groups_pw = idx2d.shape[0] // nw
    rows_pg = idx2d.shape[1] // TOPK
    mesh = plsc.VectorSubcoreMesh(core_axis_name="c", subcore_axis_name="s")

    @functools.partial(
        pl.kernel,
        out_type=jax.ShapeDtypeStruct((n_out, DH), jnp.float32),
        mesh=mesh,
        scratch_types=[
            pltpu.VMEM((groups_pw, 128), jnp.int32),
            pltpu.VMEM((groups_pw * 128 + 16,), jnp.float32),
            pltpu.VMEM((128, DH), jnp.float32),
            pltpu.VMEM((128, DH), jnp.float32),
            pltpu.VMEM((rows_pg, DH), jnp.float32),
            pltpu.VMEM((rows_pg, DH), jnp.float32),
            pltpu.SemaphoreType.DMA,
            pltpu.SemaphoreType.DMA,
        ],
    )
    def k(table_h, idx_h, w_h, out_h, idx_v, w_v, rows_a, rows_b,
          out_a, out_b, sem_a, sem_b):
        wid = lax.axis_index("s") * info.num_cores + lax.axis_index("c")
        pltpu.sync_copy(idx_h.at[pl.ds(wid * groups_pw, groups_pw)], idx_v)
        pltpu.sync_copy(w_h.at[pl.ds(wid * groups_pw * 128, groups_pw * 128)],
                        w_v.at[pl.ds(0, groups_pw * 128)])
        out_base = wid * groups_pw * rows_pg

        def combine_store(g, rows_v, out_v):
            def per_row(r, _):
                base = g * 128 + r * TOPK
                wvec = w_v[pl.ds(base, 16)]
                ws = [wvec[kk] for kk in range(TOPK)]
                for j in range(DH // 16):
                    acc = jnp.zeros((16,), jnp.float32)
                    for kk in range(TOPK):
                        acc = acc + ws[kk] * rows_v.at[r * TOPK + kk][
                            pl.ds(j * 16, 16)]
                    out_v.at[r][pl.ds(j * 16, 16)] = acc
                return _

            lax.fori_loop(0, rows_pg, per_row, None)
            pltpu.sync_copy(out_v, out_h.at[pl.ds(out_base + g * rows_pg,
                                                  rows_pg)])

        pltpu.async_copy(table_h.at[idx_v.at[0]], rows_a, sem_a)

        def pair(i, _):
            g0 = 2 * i
            g1 = g0 + 1
            pltpu.make_async_copy(table_h.at[idx_v.at[g0]], rows_a,
                                  sem_a).wait()
            pltpu.async_copy(table_h.at[idx_v.at[g1]], rows_b, sem_b)
            combine_store(g0, rows_a, out_a)
            pltpu.make_async_copy(table_h.at[idx_v.at[g1]], rows_b,
                                  sem_b).wait()

            @pl.when(i < groups_pw // 2 - 1)
            def _start_next():
                pltpu.async_copy(table_h.at[idx_v.at[g1 + 1]], rows_a, sem_a)

            combine_store(g1, rows_b, out_b)
            return _

        lax.fori_loop(0, groups_pw // 2, pair, None)

    return k(table, idx2d, wflat)


def _run_stage_a(tokens2d, keys, Wq, Wg, Wv, gq, gg, gv):
    n = tokens2d.shape[0]
    grid = n // BLK
    const = lambda shape: pl.BlockSpec(shape, lambda i: (0,) * len(shape))
    return pl.pallas_call(
        _stage_a,
        grid=(grid,),
        in_specs=[
            pl.BlockSpec((BLK, DIM), lambda i: (i, 0)),
            const((HEADS, NUM_KEYS, DH)),
            const((HEADS, NUM_KEYS, DH)),
            const((DIM, DQK * 2)),
            const((1, DIM)),
            const((DIM, DV)),
            const((1, DIM)),
            const((1, DIM)),
            const((1, DIM)),
        ],
        out_specs=[
            pl.BlockSpec((BLK, HEADS * TOPK), lambda i: (i, 0)),
            pl.BlockSpec((BLK, HEADS * TOPK), lambda i: (i, 0)),
            pl.BlockSpec((BLK, DV), lambda i: (i, 0)),
            pl.BlockSpec((BLK, DH), lambda i: (i, 0)),
        ],
        out_shape=[
            jax.ShapeDtypeStruct((n, HEADS * TOPK), jnp.int32),
            jax.ShapeDtypeStruct((n, HEADS * TOPK), jnp.float32),
            jax.ShapeDtypeStruct((n, DV), jnp.float32),
            jax.ShapeDtypeStruct((n, DH), jnp.float32),
        ],
    )(tokens2d, keys[0], keys[1], Wq, Wg.reshape(1, DIM), Wv,
      gq.reshape(1, DIM), gg.reshape(1, DIM), gv.reshape(1, DIM))


def _run_stage_c(values, tvn, gate, go, Wo):
    n = tvn.shape[0]
    grid = n // BLK
    const = lambda shape: pl.BlockSpec(shape, lambda i: (0,) * len(shape))
    return pl.pallas_call(
        _stage_c,
        grid=(grid,),
        in_specs=[
            pl.BlockSpec((BLK, DV), lambda i: (i, 0)),
            pl.BlockSpec((BLK, DV), lambda i: (i, 0)),
            pl.BlockSpec((BLK, DH), lambda i: (i, 0)),
            const((1, DV)),
            const((DV, DIM)),
        ],
        out_specs=pl.BlockSpec((BLK, DIM), lambda i: (i, 0)),
        out_shape=jax.ShapeDtypeStruct((n, DIM), jnp.float32),
    )(values, tvn, gate, go.reshape(1, DV), Wo)


def kernel(tokens, memories, keys, Wq, Wg, Wv, Wo, gq, gg, gv, go):
    b, n, _ = tokens.shape
    tok2d = tokens.reshape(b * n, DIM)
    fidx, wts, tvn, gate = _run_stage_a(tok2d, keys, Wq, Wg, Wv, gq, gg, gv)
    table = memories.reshape(-1, DH)
    idx2d = fidx.reshape(-1, 128)
    wflat = wts.reshape(-1)
    vals = _sc_gather_combine(table, idx2d, wflat)
    out = _run_stage_c(vals.reshape(b * n, DV), tvn, gate, go, Wo)
    return out.reshape(b, n, DIM)


# speedup vs baseline: 18.0406x; 1.0090x over previous
import functools

import jax
import jax.numpy as jnp
from jax import lax
from jax.experimental import pallas as pl
from jax.experimental.pallas import tpu as pltpu
from jax.experimental.pallas import tpu_sc as plsc

DIM = 1024
HEADS = 4
NUM_KEYS = 256
DQK = 512
DV = 512
TOPK = 8
DH = 128
RMS_EPS = 1.1920929e-07
IDW_EPS = 0.001
BLK = 256
TCH = 128


def _rms(x, g):
    return x / jnp.sqrt(jnp.mean(x * x, axis=-1, keepdims=True) + RMS_EPS) * g


def _top8(d):
    iota = lax.broadcasted_iota(jnp.int32, d.shape, 0).astype(jnp.float32)
    work = d
    vals, poss = [], []
    for _ in range(TOPK):
        m = jnp.min(work, axis=0, keepdims=True)
        pos = jnp.min(jnp.where(work == m, iota, jnp.float32(512.0)),
                      axis=0, keepdims=True)
        work = jnp.where(iota == pos, jnp.float32(jnp.inf), work)
        vals.append(m)
        poss.append(pos)
    return (jnp.concatenate(poss, axis=0).astype(jnp.int32),
            jnp.concatenate(vals, axis=0))


def _sel8(arr, sel):
    out = jnp.zeros(sel.shape, arr.dtype)
    for i in range(TOPK):
        out = jnp.where(sel == i, arr[i:i + 1, :], out)
    return out


def _stage_a(tok, k1, k2, wq, wgr, wv, gq, gg, gv,
             fidx_o, wts_o, tvn_o, gate_o):
    x = tok[...]
    xq = _rms(x, gq[...])
    q = jnp.dot(xq, wq[...], preferred_element_type=jnp.float32)
    dn_t = (((1,), (1,)), ((), ()))
    for h in range(HEADS):
        k1h = k1[h]
        k2h = k2[h]
        ks1 = jnp.sum(k1h * k1h, axis=-1, keepdims=True)
        ks2 = jnp.sum(k2h * k2h, axis=-1, keepdims=True)
        for c in range(0, q.shape[0], TCH):
            sl = slice(c, c + TCH)
            q1 = q[sl, h * DH:(h + 1) * DH]
            q2 = q[sl, DQK + h * DH:DQK + (h + 1) * DH]
            c1 = lax.dot_general(k1h, q1, dn_t,
                                 preferred_element_type=jnp.float32)
            c2 = lax.dot_general(k2h, q2, dn_t,
                                 preferred_element_type=jnp.float32)
            qs1 = jnp.transpose(jnp.sum(q1 * q1, axis=-1, keepdims=True))
            qs2 = jnp.transpose(jnp.sum(q2 * q2, axis=-1, keepdims=True))
            idx1, s1 = _top8((qs1 + ks1) - 2.0 * c1 + IDW_EPS)
            idx2, s2 = _top8((qs2 + ks2) - 2.0 * c2 + IDW_EPS)
            prod = (s1[:, None, :] * s2[None, :, :]).reshape(
                TOPK * TOPK, -1)
            pos, pval = _top8(prod)
            g1 = _sel8(idx1, lax.shift_right_logical(pos, 3))
            g2 = _sel8(idx2, jnp.bitwise_and(pos, jnp.int32(7)))
            w = 1.0 / pval
            w = w / jnp.sum(w, axis=0, keepdims=True)
            fidx_o[sl, h * TOPK:(h + 1) * TOPK] = jnp.transpose(
                (g1 * NUM_KEYS + g2) * HEADS + h)
            wts_o[sl, h * TOPK:(h + 1) * TOPK] = jnp.transpose(w)

    xg = _rms(x, gg[...])
    gate = jax.nn.sigmoid(jnp.sum(xg * wgr[...], axis=-1, keepdims=True))
    xv = _rms(x, gv[...])
    tv = jnp.dot(xv, wv[...], preferred_element_type=jnp.float32)
    for h in range(HEADS):
        th = tv[:, h * DH:(h + 1) * DH]
        mu = jnp.mean(th, axis=-1, keepdims=True)
        ctr = th - mu
        std = jnp.sqrt(jnp.sum(ctr * ctr, axis=-1, keepdims=True) / (DH - 1))
        tvn_o[:, h * DH:(h + 1) * DH] = ctr / jnp.maximum(std, 1e-10)
    gate_o[...] = jnp.broadcast_to(gate, (gate.shape[0], DH))


def _stage_c(vals, tvn, gate, go, wo, out_o):
    g = gate[:, 0:1]
    o = tvn[...] + g * (vals[...] - tvn[...])
    on = _rms(o, go[...])
    out_o[...] = jnp.dot(on, wo[...], preferred_element_type=jnp.float32)


def _sc_gather_combine(table, idx2d, wflat):
    info = plsc.get_sparse_core_info()
    nw = info.num_cores * info.num_subcores
    n_out = idx2d.shape[0] * idx2d.shape[1] // TOPK
    groups_pw = idx2d.shape[0] // nw
    rows_pg = idx2d.shape[1] // TOPK
    mesh = plsc.VectorSubcoreMesh(core_axis_name="c", subcore_axis_name="s")

    @functools.partial(
        pl.kernel,
        out_type=jax.ShapeDtypeStruct((n_out, DH), jnp.float32),
        mesh=mesh,
        scratch_types=[
            pltpu.VMEM((groups_pw, 128), jnp.int32),
            pltpu.VMEM((groups_pw * 128 + 16,), jnp.float32),
            pltpu.VMEM((128, DH), jnp.float32),
            pltpu.VMEM((128, DH), jnp.float32),
            pltpu.VMEM((rows_pg, DH), jnp.float32),
            pltpu.VMEM((rows_pg, DH), jnp.float32),
            pltpu.SemaphoreType.DMA,
            pltpu.SemaphoreType.DMA,
        ],
    )
    def k(table_h, idx_h, w_h, out_h, idx_v, w_v, rows_a, rows_b,
          out_a, out_b, sem_a, sem_b):
        wid = lax.axis_index("s") * info.num_cores + lax.axis_index("c")
        pltpu.sync_copy(idx_h.at[pl.ds(wid * groups_pw, groups_pw)], idx_v)
        pltpu.sync_copy(w_h.at[pl.ds(wid * groups_pw * 128, groups_pw * 128)],
                        w_v.at[pl.ds(0, groups_pw * 128)])
        out_base = wid * groups_pw * rows_pg

        def combine_store(g, rows_v, out_v):
            def per_row(r, _):
                base = g * 128 + r * TOPK
                wvec = w_v[pl.ds(base, 16)]
                ws = [wvec[kk] for kk in range(TOPK)]
                for j in range(DH // 16):
                    acc = jnp.zeros((16,), jnp.float32)
                    for kk in range(TOPK):
                        acc = acc + ws[kk] * rows_v.at[r * TOPK + kk][
                            pl.ds(j * 16, 16)]
                    out_v.at[r][pl.ds(j * 16, 16)] = acc
                return _

            lax.fori_loop(0, rows_pg, per_row, None)
            pltpu.sync_copy(out_v, out_h.at[pl.ds(out_base + g * rows_pg,
                                                  rows_pg)])

        pltpu.async_copy(table_h.at[idx_v.at[0]], rows_a, sem_a)

        def pair(i, _):
            g0 = 2 * i
            g1 = g0 + 1
            pltpu.make_async_copy(table_h.at[idx_v.at[g0]], rows_a,
                                  sem_a).wait()
            pltpu.async_copy(table_h.at[idx_v.at[g1]], rows_b, sem_b)
            combine_store(g0, rows_a, out_a)
            pltpu.make_async_copy(table_h.at[idx_v.at[g1]], rows_b,
                                  sem_b).wait()

            @pl.when(i < groups_pw // 2 - 1)
            def _start_next():
                pltpu.async_copy(table_h.at[idx_v.at[g1 + 1]], rows_a, sem_a)

            combine_store(g1, rows_b, out_b)
            return _

        lax.fori_loop(0, groups_pw // 2, pair, None)

    return k(table, idx2d, wflat)


def _run_stage_a(tokens2d, keys, Wq, Wg, Wv, gq, gg, gv):
    n = tokens2d.shape[0]
    grid = n // BLK
    const = lambda shape: pl.BlockSpec(shape, lambda i: (0,) * len(shape))
    return pl.pallas_call(
        _stage_a,
        grid=(grid,),
        in_specs=[
            pl.BlockSpec((BLK, DIM), lambda i: (i, 0)),
            const((HEADS, NUM_KEYS, DH)),
            const((HEADS, NUM_KEYS, DH)),
            const((DIM, DQK * 2)),
            const((1, DIM)),
            const((DIM, DV)),
            const((1, DIM)),
            const((1, DIM)),
            const((1, DIM)),
        ],
        out_specs=[
            pl.BlockSpec((BLK, HEADS * TOPK), lambda i: (i, 0)),
            pl.BlockSpec((BLK, HEADS * TOPK), lambda i: (i, 0)),
            pl.BlockSpec((BLK, DV), lambda i: (i, 0)),
            pl.BlockSpec((BLK, DH), lambda i: (i, 0)),
        ],
        out_shape=[
            jax.ShapeDtypeStruct((n, HEADS * TOPK), jnp.int32),
            jax.ShapeDtypeStruct((n, HEADS * TOPK), jnp.float32),
            jax.ShapeDtypeStruct((n, DV), jnp.float32),
            jax.ShapeDtypeStruct((n, DH), jnp.float32),
        ],
    )(tokens2d, keys[0], keys[1], Wq, Wg.reshape(1, DIM), Wv,
      gq.reshape(1, DIM), gg.reshape(1, DIM), gv.reshape(1, DIM))


def _run_stage_c(values, tvn, gate, go, Wo):
    n = tvn.shape[0]
    grid = n // BLK
    const = lambda shape: pl.BlockSpec(shape, lambda i: (0,) * len(shape))
    return pl.pallas_call(
        _stage_c,
        grid=(grid,),
        in_specs=[
            pl.BlockSpec((BLK, DV), lambda i: (i, 0)),
            pl.BlockSpec((BLK, DV), lambda i: (i, 0)),
            pl.BlockSpec((BLK, DH), lambda i: (i, 0)),
            const((1, DV)),
            const((DV, DIM)),
        ],
        out_specs=pl.BlockSpec((BLK, DIM), lambda i: (i, 0)),
        out_shape=jax.ShapeDtypeStruct((n, DIM), jnp.float32),
    )(values, tvn, gate, go.reshape(1, DV), Wo)


def kernel(tokens, memories, keys, Wq, Wg, Wv, Wo, gq, gg, gv, go):
    b, n, _ = tokens.shape
    tok2d = tokens.reshape(b * n, DIM)
    table = memories.reshape(-1, DH)
    nsplit = 4
    ntok = (b * n) // nsplit
    parts = []
    for s in range(nsplit):
        t2 = tok2d[s * ntok:(s + 1) * ntok]
        fidx, wts, tvn, gate = _run_stage_a(t2, keys, Wq, Wg, Wv, gq, gg, gv)
        vals = _sc_gather_combine(table, fidx.reshape(-1, 128),
                                  wts.reshape(-1))
        parts.append(_run_stage_c(vals.reshape(ntok, DV), tvn, gate, go, Wo))
    return jnp.concatenate(parts, axis=0).reshape(b, n, DIM)


# speedup vs baseline: 18.1700x; 1.0072x over previous
import functools

import jax
import jax.numpy as jnp
from jax import lax
from jax.experimental import pallas as pl
from jax.experimental.pallas import tpu as pltpu
from jax.experimental.pallas import tpu_sc as plsc

DIM = 1024
HEADS = 4
NUM_KEYS = 256
DQK = 512
DV = 512
TOPK = 8
DH = 128
RMS_EPS = 1.1920929e-07
IDW_EPS = 0.001
BLK = 256
TCH = 128


def _rms(x, g):
    return x / jnp.sqrt(jnp.mean(x * x, axis=-1, keepdims=True) + RMS_EPS) * g


def _top8(d):
    iota = lax.broadcasted_iota(jnp.int32, d.shape, 0).astype(jnp.float32)
    work = d
    vals, poss = [], []
    for _ in range(TOPK):
        m = jnp.min(work, axis=0, keepdims=True)
        pos = jnp.min(jnp.where(work == m, iota, jnp.float32(512.0)),
                      axis=0, keepdims=True)
        work = jnp.where(iota == pos, jnp.float32(jnp.inf), work)
        vals.append(m)
        poss.append(pos)
    return (jnp.concatenate(poss, axis=0).astype(jnp.int32),
            jnp.concatenate(vals, axis=0))


def _sel8(arr, sel):
    out = jnp.zeros(sel.shape, arr.dtype)
    for i in range(TOPK):
        out = jnp.where(sel == i, arr[i:i + 1, :], out)
    return out


def _stage_a(tok, k1, k2, wq, wgr, wv, gq, gg, gv,
             fidx_o, wts_o, tvn_o, gate_o):
    x = tok[...]
    xq = _rms(x, gq[...])
    q = jnp.dot(xq, wq[...], preferred_element_type=jnp.float32)
    dn_t = (((1,), (1,)), ((), ()))
    for h in range(HEADS):
        k1h = k1[h]
        k2h = k2[h]
        ks1 = jnp.sum(k1h * k1h, axis=-1, keepdims=True)
        ks2 = jnp.sum(k2h * k2h, axis=-1, keepdims=True)
        for c in range(0, q.shape[0], TCH):
            sl = slice(c, c + TCH)
            q1 = q[sl, h * DH:(h + 1) * DH]
            q2 = q[sl, DQK + h * DH:DQK + (h + 1) * DH]
            c1 = lax.dot_general(k1h, q1, dn_t,
                                 preferred_element_type=jnp.float32)
            c2 = lax.dot_general(k2h, q2, dn_t,
                                 preferred_element_type=jnp.float32)
            qs1 = jnp.transpose(jnp.sum(q1 * q1, axis=-1, keepdims=True))
            qs2 = jnp.transpose(jnp.sum(q2 * q2, axis=-1, keepdims=True))
            idx1, s1 = _top8((qs1 + ks1) - 2.0 * c1 + IDW_EPS)
            idx2, s2 = _top8((qs2 + ks2) - 2.0 * c2 + IDW_EPS)
            prod = (s1[:, None, :] * s2[None, :, :]).reshape(
                TOPK * TOPK, -1)
            pos, pval = _top8(prod)
            g1 = _sel8(idx1, lax.shift_right_logical(pos, 3))
            g2 = _sel8(idx2, jnp.bitwise_and(pos, jnp.int32(7)))
            w = 1.0 / pval
            w = w / jnp.sum(w, axis=0, keepdims=True)
            fidx_o[sl, h * TOPK:(h + 1) * TOPK] = jnp.transpose(
                (g1 * NUM_KEYS + g2) * HEADS + h)
            wts_o[sl, h * TOPK:(h + 1) * TOPK] = jnp.transpose(w)

    xg = _rms(x, gg[...])
    gate = jax.nn.sigmoid(jnp.sum(xg * wgr[...], axis=-1, keepdims=True))
    xv = _rms(x, gv[...])
    tv = jnp.dot(xv, wv[...], preferred_element_type=jnp.float32)
    for h in range(HEADS):
        th = tv[:, h * DH:(h + 1) * DH]
        mu = jnp.mean(th, axis=-1, keepdims=True)
        ctr = th - mu
        std = jnp.sqrt(jnp.sum(ctr * ctr, axis=-1, keepdims=True) / (DH - 1))
        tvn_o[:, h * DH:(h + 1) * DH] = ctr / jnp.maximum(std, 1e-10)
    gate_o[...] = jnp.broadcast_to(gate, (gate.shape[0], DH))


def _stage_c(vals, tvn, gate, go, wo, out_o):
    g = gate[:, 0:1]
    o = tvn[...] + g * (vals[...] - tvn[...])
    on = _rms(o, go[...])
    out_o[...] = jnp.dot(on, wo[...], preferred_element_type=jnp.float32)


def _sc_gather_combine(table, idx2d, wflat):
    info = plsc.get_sparse_core_info()
    nw = info.num_cores * info.num_subcores
    n_out = idx2d.shape[0] * idx2d.shape[1] // TOPK
    groups_pw = idx2d.shape[0] // nw
    rows_pg = idx2d.shape[1] // TOPK
    mesh = plsc.VectorSubcoreMesh(core_axis_name="c", subcore_axis_name="s")

    @functools.partial(
        pl.kernel,
        out_type=jax.ShapeDtypeStruct((n_out, DH), jnp.float32),
        mesh=mesh,
        scratch_types=[
            pltpu.VMEM((groups_pw, 128), jnp.int32),
            pltpu.VMEM((groups_pw * 128 + 16,), jnp.float32),
            pltpu.VMEM((128, DH), jnp.float32),
            pltpu.VMEM((128, DH), jnp.float32),
            pltpu.VMEM((rows_pg, DH), jnp.float32),
            pltpu.VMEM((rows_pg, DH), jnp.float32),
            pltpu.SemaphoreType.DMA,
            pltpu.SemaphoreType.DMA,
        ],
    )
    def k(table_h, idx_h, w_h, out_h, idx_v, w_v, rows_a, rows_b,
          out_a, out_b, sem_a, sem_b):
        wid = lax.axis_index("s") * info.num_cores + lax.axis_index("c")
        pltpu.sync_copy(idx_h.at[pl.ds(wid * groups_pw, groups_pw)], idx_v)
        pltpu.sync_copy(w_h.at[pl.ds(wid * groups_pw * 128, groups_pw * 128)],
                        w_v.at[pl.ds(0, groups_pw * 128)])
        out_base = wid * groups_pw * rows_pg

        def combine_store(g, rows_v, out_v):
            def per_row(r, _):
                base = g * 128 + r * TOPK
                wvec = w_v[pl.ds(base, 16)]
                ws = [wvec[kk] for kk in range(TOPK)]
                for j in range(DH // 16):
                    acc = jnp.zeros((16,), jnp.float32)
                    for kk in range(TOPK):
                        acc = acc + ws[kk] * rows_v.at[r * TOPK + kk][
                            pl.ds(j * 16, 16)]
                    out_v.at[r][pl.ds(j * 16, 16)] = acc
                return _

            lax.fori_loop(0, rows_pg, per_row, None)
            pltpu.sync_copy(out_v, out_h.at[pl.ds(out_base + g * rows_pg,
                                                  rows_pg)])

        pltpu.async_copy(table_h.at[idx_v.at[0]], rows_a, sem_a)

        def pair(i, _):
            g0 = 2 * i
            g1 = g0 + 1
            pltpu.make_async_copy(table_h.at[idx_v.at[g0]], rows_a,
                                  sem_a).wait()
            pltpu.async_copy(table_h.at[idx_v.at[g1]], rows_b, sem_b)
            combine_store(g0, rows_a, out_a)
            pltpu.make_async_copy(table_h.at[idx_v.at[g1]], rows_b,
                                  sem_b).wait()

            @pl.when(i < groups_pw // 2 - 1)
            def _start_next():
                pltpu.async_copy(table_h.at[idx_v.at[g1 + 1]], rows_a, sem_a)

            combine_store(g1, rows_b, out_b)
            return _

        lax.fori_loop(0, groups_pw // 2, pair, None)

    return k(table, idx2d, wflat)


def _run_stage_a(tokens2d, keys, Wq, Wg, Wv, gq, gg, gv):
    n = tokens2d.shape[0]
    grid = n // BLK
    const = lambda shape: pl.BlockSpec(shape, lambda i: (0,) * len(shape))
    return pl.pallas_call(
        _stage_a,
        grid=(grid,),
        in_specs=[
            pl.BlockSpec((BLK, DIM), lambda i: (i, 0)),
            const((HEADS, NUM_KEYS, DH)),
            const((HEADS, NUM_KEYS, DH)),
            const((DIM, DQK * 2)),
            const((1, DIM)),
            const((DIM, DV)),
            const((1, DIM)),
            const((1, DIM)),
            const((1, DIM)),
        ],
        out_specs=[
            pl.BlockSpec((BLK, HEADS * TOPK), lambda i: (i, 0)),
            pl.BlockSpec((BLK, HEADS * TOPK), lambda i: (i, 0)),
            pl.BlockSpec((BLK, DV), lambda i: (i, 0)),
            pl.BlockSpec((BLK, DH), lambda i: (i, 0)),
        ],
        out_shape=[
            jax.ShapeDtypeStruct((n, HEADS * TOPK), jnp.int32),
            jax.ShapeDtypeStruct((n, HEADS * TOPK), jnp.float32),
            jax.ShapeDtypeStruct((n, DV), jnp.float32),
            jax.ShapeDtypeStruct((n, DH), jnp.float32),
        ],
    )(tokens2d, keys[0], keys[1], Wq, Wg.reshape(1, DIM), Wv,
      gq.reshape(1, DIM), gg.reshape(1, DIM), gv.reshape(1, DIM))


def _run_stage_c(values, tvn, gate, go, Wo):
    n = tvn.shape[0]
    grid = n // BLK
    const = lambda shape: pl.BlockSpec(shape, lambda i: (0,) * len(shape))
    return pl.pallas_call(
        _stage_c,
        grid=(grid,),
        in_specs=[
            pl.BlockSpec((BLK, DV), lambda i: (i, 0)),
            pl.BlockSpec((BLK, DV), lambda i: (i, 0)),
            pl.BlockSpec((BLK, DH), lambda i: (i, 0)),
            const((1, DV)),
            const((DV, DIM)),
        ],
        out_specs=pl.BlockSpec((BLK, DIM), lambda i: (i, 0)),
        out_shape=jax.ShapeDtypeStruct((n, DIM), jnp.float32),
    )(values, tvn, gate, go.reshape(1, DV), Wo)


def kernel(tokens, memories, keys, Wq, Wg, Wv, Wo, gq, gg, gv, go):
    b, n, _ = tokens.shape
    tok2d = tokens.reshape(b * n, DIM)
    table = memories.reshape(-1, DH)
    nsplit = 2
    ntok = (b * n) // nsplit
    parts = []
    for s in range(nsplit):
        t2 = tok2d[s * ntok:(s + 1) * ntok]
        fidx, wts, tvn, gate = _run_stage_a(t2, keys, Wq, Wg, Wv, gq, gg, gv)
        vals = _sc_gather_combine(table, fidx.reshape(-1, 128),
                                  wts.reshape(-1))
        parts.append(_run_stage_c(vals.reshape(ntok, DV), tvn, gate, go, Wo))
    return jnp.concatenate(parts, axis=0).reshape(b, n, DIM)


# speedup vs baseline: 19.0703x; 1.0496x over previous
import functools

import jax
import jax.numpy as jnp
from jax import lax
from jax.experimental import pallas as pl
from jax.experimental.pallas import tpu as pltpu
from jax.experimental.pallas import tpu_sc as plsc

DIM = 1024
HEADS = 4
NUM_KEYS = 256
DQK = 512
DV = 512
TOPK = 8
DH = 128
RMS_EPS = 1.1920929e-07
IDW_EPS = 0.001
BLK = 256
TCH = 128


def _rms(x, g):
    return x / jnp.sqrt(jnp.mean(x * x, axis=-1, keepdims=True) + RMS_EPS) * g


def _top8(d):
    iota = lax.broadcasted_iota(jnp.int32, d.shape, 0).astype(jnp.float32)
    work = d
    vals, poss = [], []
    for _ in range(TOPK):
        m = jnp.min(work, axis=0, keepdims=True)
        pos = jnp.min(jnp.where(work == m, iota, jnp.float32(512.0)),
                      axis=0, keepdims=True)
        work = jnp.where(iota == pos, jnp.float32(jnp.inf), work)
        vals.append(m)
        poss.append(pos)
    return (jnp.concatenate(poss, axis=0).astype(jnp.int32),
            jnp.concatenate(vals, axis=0))


def _sel8(arr, sel):
    out = jnp.zeros(sel.shape, arr.dtype)
    for i in range(TOPK):
        out = jnp.where(sel == i, arr[i:i + 1, :], out)
    return out


def _stage_a(tok, k1, k2, wq, wgr, wv, gq, gg, gv,
             fidx_o, wts_o, tvn_o, gate_o):
    x = tok[...]
    xq = _rms(x, gq[...])
    q = jnp.dot(xq, wq[...], preferred_element_type=jnp.float32)
    dn_t = (((1,), (1,)), ((), ()))
    for h in range(HEADS):
        k1h = k1[h]
        k2h = k2[h]
        ks1 = jnp.sum(k1h * k1h, axis=-1, keepdims=True)
        ks2 = jnp.sum(k2h * k2h, axis=-1, keepdims=True)
        for c in range(0, q.shape[0], TCH):
            sl = slice(c, c + TCH)
            q1 = q[sl, h * DH:(h + 1) * DH]
            q2 = q[sl, DQK + h * DH:DQK + (h + 1) * DH]
            c1 = lax.dot_general(k1h, q1, dn_t,
                                 preferred_element_type=jnp.float32)
            c2 = lax.dot_general(k2h, q2, dn_t,
                                 preferred_element_type=jnp.float32)
            qs1 = jnp.transpose(jnp.sum(q1 * q1, axis=-1, keepdims=True))
            qs2 = jnp.transpose(jnp.sum(q2 * q2, axis=-1, keepdims=True))
            idx1, s1 = _top8((qs1 + ks1) - 2.0 * c1 + IDW_EPS)
            idx2, s2 = _top8((qs2 + ks2) - 2.0 * c2 + IDW_EPS)
            prod = (s1[:, None, :] * s2[None, :, :]).reshape(
                TOPK * TOPK, -1)
            pos, pval = _top8(prod)
            g1 = _sel8(idx1, lax.shift_right_logical(pos, 3))
            g2 = _sel8(idx2, jnp.bitwise_and(pos, jnp.int32(7)))
            w = 1.0 / pval
            w = w / jnp.sum(w, axis=0, keepdims=True)
            fidx_o[sl, h * TOPK:(h + 1) * TOPK] = jnp.transpose(
                (g1 * NUM_KEYS + g2) * HEADS + h)
            wts_o[sl, h * TOPK:(h + 1) * TOPK] = jnp.transpose(w)

    xg = _rms(x, gg[...])
    gate = jax.nn.sigmoid(jnp.sum(xg * wgr[...], axis=-1, keepdims=True))
    xv = _rms(x, gv[...])
    tv = jnp.dot(xv, wv[...], preferred_element_type=jnp.float32)
    for h in range(HEADS):
        th = tv[:, h * DH:(h + 1) * DH]
        mu = jnp.mean(th, axis=-1, keepdims=True)
        ctr = th - mu
        std = jnp.sqrt(jnp.sum(ctr * ctr, axis=-1, keepdims=True) / (DH - 1))
        tvn_o[:, h * DH:(h + 1) * DH] = ctr / jnp.maximum(std, 1e-10)
    gate_o[...] = jnp.broadcast_to(gate, (gate.shape[0], DH))


def _stage_c(vals, tvn, gate, go, wo, out_o):
    g = gate[:, 0:1]
    o = tvn[...] + g * (vals[...] - tvn[...])
    on = _rms(o, go[...])
    out_o[...] = jnp.dot(on, wo[...], preferred_element_type=jnp.float32)


def _sc_gather_combine(table, idx, wts):
    info = plsc.get_sparse_core_info()
    nw = info.num_cores * info.num_subcores
    ntok = idx.shape[0]
    trows_pw = ntok // nw
    ngrp = trows_pw // 4
    mesh = plsc.VectorSubcoreMesh(core_axis_name="c", subcore_axis_name="s")

    @functools.partial(
        pl.kernel,
        out_type=jax.ShapeDtypeStruct((ntok, DV), jnp.float32),
        mesh=mesh,
        scratch_types=[
            pltpu.VMEM((trows_pw, 32), jnp.int32),
            pltpu.VMEM((trows_pw, 32), jnp.float32),
            pltpu.VMEM((128, DH), jnp.float32),
            pltpu.VMEM((128, DH), jnp.float32),
            pltpu.VMEM((4, DV), jnp.float32),
            pltpu.VMEM((4, DV), jnp.float32),
            pltpu.SemaphoreType.DMA,
            pltpu.SemaphoreType.DMA,
        ],
    )
    def k(table_h, idx_h, w_h, out_h, idx_v, w_v, rows_a, rows_b,
          out_a, out_b, sem_a, sem_b):
        wid = lax.axis_index("s") * info.num_cores + lax.axis_index("c")
        tok0 = wid * trows_pw
        pltpu.sync_copy(idx_h.at[pl.ds(tok0, trows_pw)], idx_v)
        pltpu.sync_copy(w_h.at[pl.ds(tok0, trows_pw)], w_v)

        def gather(g, rows_v, sem):
            for i in range(4):
                pltpu.async_copy(table_h.at[idx_v.at[4 * g + i]],
                                 rows_v.at[pl.ds(i * 32, 32)], sem)

        def gather_wait(g, rows_v, sem):
            for i in range(4):
                pltpu.make_async_copy(table_h.at[idx_v.at[4 * g + i]],
                                      rows_v.at[pl.ds(i * 32, 32)],
                                      sem).wait()

        def combine_store(g, rows_v, out_v):
            def per_tok(i, _):
                wlo = w_v.at[4 * g + i][pl.ds(0, 16)]
                whi = w_v.at[4 * g + i][pl.ds(16, 16)]
                for h in range(HEADS):
                    wsrc = wlo if h < 2 else whi
                    lane0 = (h % 2) * TOPK
                    ws = [wsrc[lane0 + kk] for kk in range(TOPK)]
                    for j in range(DH // 16):
                        acc = jnp.zeros((16,), jnp.float32)
                        for kk in range(TOPK):
                            acc = acc + ws[kk] * rows_v.at[
                                i * 32 + h * TOPK + kk][pl.ds(j * 16, 16)]
                        out_v.at[i][pl.ds(h * DH + j * 16, 16)] = acc
                return _

            lax.fori_loop(0, 4, per_tok, None)
            pltpu.sync_copy(out_v, out_h.at[pl.ds(tok0 + 4 * g, 4)])

        gather(0, rows_a, sem_a)

        def pair(i, _):
            g0 = 2 * i
            g1 = g0 + 1
            gather_wait(g0, rows_a, sem_a)
            gather(g1, rows_b, sem_b)
            combine_store(g0, rows_a, out_a)
            gather_wait(g1, rows_b, sem_b)

            @pl.when(i < ngrp // 2 - 1)
            def _start_next():
                gather(g1 + 1, rows_a, sem_a)

            combine_store(g1, rows_b, out_b)
            return _

        lax.fori_loop(0, ngrp // 2, pair, None)

    return k(table, idx, wts)


def _run_stage_a(tokens2d, keys, Wq, Wg, Wv, gq, gg, gv):
    n = tokens2d.shape[0]
    grid = n // BLK
    const = lambda shape: pl.BlockSpec(shape, lambda i: (0,) * len(shape))
    return pl.pallas_call(
        _stage_a,
        grid=(grid,),
        in_specs=[
            pl.BlockSpec((BLK, DIM), lambda i: (i, 0)),
            const((HEADS, NUM_KEYS, DH)),
            const((HEADS, NUM_KEYS, DH)),
            const((DIM, DQK * 2)),
            const((1, DIM)),
            const((DIM, DV)),
            const((1, DIM)),
            const((1, DIM)),
            const((1, DIM)),
        ],
        out_specs=[
            pl.BlockSpec((BLK, HEADS * TOPK), lambda i: (i, 0)),
            pl.BlockSpec((BLK, HEADS * TOPK), lambda i: (i, 0)),
            pl.BlockSpec((BLK, DV), lambda i: (i, 0)),
            pl.BlockSpec((BLK, DH), lambda i: (i, 0)),
        ],
        out_shape=[
            jax.ShapeDtypeStruct((n, HEADS * TOPK), jnp.int32),
            jax.ShapeDtypeStruct((n, HEADS * TOPK), jnp.float32),
            jax.ShapeDtypeStruct((n, DV), jnp.float32),
            jax.ShapeDtypeStruct((n, DH), jnp.float32),
        ],
    )(tokens2d, keys[0], keys[1], Wq, Wg.reshape(1, DIM), Wv,
      gq.reshape(1, DIM), gg.reshape(1, DIM), gv.reshape(1, DIM))


def _run_stage_c(values, tvn, gate, go, Wo):
    n = tvn.shape[0]
    grid = n // BLK
    const = lambda shape: pl.BlockSpec(shape, lambda i: (0,) * len(shape))
    return pl.pallas_call(
        _stage_c,
        grid=(grid,),
        in_specs=[
            pl.BlockSpec((BLK, DV), lambda i: (i, 0)),
            pl.BlockSpec((BLK, DV), lambda i: (i, 0)),
            pl.BlockSpec((BLK, DH), lambda i: (i, 0)),
            const((1, DV)),
            const((DV, DIM)),
        ],
        out_specs=pl.BlockSpec((BLK, DIM), lambda i: (i, 0)),
        out_shape=jax.ShapeDtypeStruct((n, DIM), jnp.float32),
    )(values, tvn, gate, go.reshape(1, DV), Wo)


def kernel(tokens, memories, keys, Wq, Wg, Wv, Wo, gq, gg, gv, go):
    b, n, _ = tokens.shape
    tok2d = tokens.reshape(b * n, DIM)
    table = memories.reshape(-1, DH)
    nsplit = 2
    ntok = (b * n) // nsplit
    parts = []
    for s in range(nsplit):
        t2 = tok2d[s * ntok:(s + 1) * ntok]
        fidx, wts, tvn, gate = _run_stage_a(t2, keys, Wq, Wg, Wv, gq, gg, gv)
        vals = _sc_gather_combine(table, fidx, wts)
        parts.append(_run_stage_c(vals, tvn, gate, go, Wo))
    return jnp.concatenate(parts, axis=0).reshape(b, n, DIM)


# speedup vs baseline: 21.1401x; 1.1085x over previous
import functools

import jax
import jax.numpy as jnp
from jax import lax
from jax.experimental import pallas as pl
from jax.experimental.pallas import tpu as pltpu
from jax.experimental.pallas import tpu_sc as plsc

DIM = 1024
HEADS = 4
NUM_KEYS = 256
DQK = 512
DV = 512
TOPK = 8
DH = 128
RMS_EPS = 1.1920929e-07
IDW_EPS = 0.001
BLK = 256
TCH = 128


def _rms(x, g):
    return x / jnp.sqrt(jnp.mean(x * x, axis=-1, keepdims=True) + RMS_EPS) * g


def _top8(d):
    iota = lax.broadcasted_iota(jnp.int32, d.shape, 0).astype(jnp.float32)
    work = d
    vals, poss = [], []
    for _ in range(TOPK):
        m = jnp.min(work, axis=0, keepdims=True)
        pos = jnp.min(jnp.where(work == m, iota, jnp.float32(512.0)),
                      axis=0, keepdims=True)
        work = jnp.where(iota == pos, jnp.float32(jnp.inf), work)
        vals.append(m)
        poss.append(pos)
    return (jnp.concatenate(poss, axis=0).astype(jnp.int32),
            jnp.concatenate(vals, axis=0))


def _sel8(arr, sel):
    out = jnp.zeros(sel.shape, arr.dtype)
    for i in range(TOPK):
        out = jnp.where(sel == i, arr[i:i + 1, :], out)
    return out


def _stage_a(tok, k1, k2, wq, wgr, wv, gq, gg, gv,
             fidx_o, wts_o, tvn_o, gate_o):
    x = tok[...]
    xq = _rms(x, gq[...])
    q = jnp.dot(xq, wq[...], preferred_element_type=jnp.float32)
    dn_t = (((1,), (1,)), ((), ()))
    for h in range(HEADS):
        k1h = k1[h]
        k2h = k2[h]
        ks1 = jnp.sum(k1h * k1h, axis=-1, keepdims=True)
        ks2 = jnp.sum(k2h * k2h, axis=-1, keepdims=True)
        for c in range(0, q.shape[0], TCH):
            sl = slice(c, c + TCH)
            q1 = q[sl, h * DH:(h + 1) * DH]
            q2 = q[sl, DQK + h * DH:DQK + (h + 1) * DH]
            c1 = lax.dot_general(k1h, q1, dn_t,
                                 preferred_element_type=jnp.float32)
            c2 = lax.dot_general(k2h, q2, dn_t,
                                 preferred_element_type=jnp.float32)
            qs1 = jnp.transpose(jnp.sum(q1 * q1, axis=-1, keepdims=True))
            qs2 = jnp.transpose(jnp.sum(q2 * q2, axis=-1, keepdims=True))
            idx1, s1 = _top8((qs1 + ks1) - 2.0 * c1 + IDW_EPS)
            idx2, s2 = _top8((qs2 + ks2) - 2.0 * c2 + IDW_EPS)
            prod = (s1[:, None, :] * s2[None, :, :]).reshape(
                TOPK * TOPK, -1)
            pos, pval = _top8(prod)
            g1 = _sel8(idx1, lax.shift_right_logical(pos, 3))
            g2 = _sel8(idx2, jnp.bitwise_and(pos, jnp.int32(7)))
            w = 1.0 / pval
            w = w / jnp.sum(w, axis=0, keepdims=True)
            fidx_o[sl, h * TOPK:(h + 1) * TOPK] = jnp.transpose(
                (g1 * NUM_KEYS + g2) * HEADS + h)
            wts_o[sl, h * TOPK:(h + 1) * TOPK] = jnp.transpose(w)

    xg = _rms(x, gg[...])
    gate = jax.nn.sigmoid(jnp.sum(xg * wgr[...], axis=-1, keepdims=True))
    xv = _rms(x, gv[...])
    tv = jnp.dot(xv, wv[...], preferred_element_type=jnp.float32)
    for h in range(HEADS):
        th = tv[:, h * DH:(h + 1) * DH]
        mu = jnp.mean(th, axis=-1, keepdims=True)
        ctr = th - mu
        std = jnp.sqrt(jnp.sum(ctr * ctr, axis=-1, keepdims=True) / (DH - 1))
        tvn_o[:, h * DH:(h + 1) * DH] = ctr / jnp.maximum(std, 1e-10)
    gate_o[...] = jnp.broadcast_to(gate, (gate.shape[0], DH))


def _stage_c(vals, tvn, gate, go, wo, out_o):
    g = gate[:, 0:1]
    o = tvn[...] + g * (vals[...] - tvn[...])
    on = _rms(o, go[...])
    out_o[...] = jnp.dot(on, wo[...], preferred_element_type=jnp.float32)


def _sc_gather_combine(table, idx, wts):
    info = plsc.get_sparse_core_info()
    nw = info.num_cores * info.num_subcores
    ntok = idx.shape[0]
    trows_pw = ntok // nw
    ngrp = trows_pw // 4
    mesh = plsc.VectorSubcoreMesh(core_axis_name="c", subcore_axis_name="s")

    @functools.partial(
        pl.kernel,
        out_type=jax.ShapeDtypeStruct((ntok, DV), jnp.float32),
        mesh=mesh,
        scratch_types=[
            pltpu.VMEM((trows_pw, 32), jnp.int32),
            pltpu.VMEM((trows_pw, 32), jnp.float32),
            pltpu.VMEM((128, DH), jnp.float32),
            pltpu.VMEM((128, DH), jnp.float32),
            pltpu.VMEM((4, DV), jnp.float32),
            pltpu.VMEM((4, DV), jnp.float32),
            pltpu.SemaphoreType.DMA,
            pltpu.SemaphoreType.DMA,
        ],
    )
    def k(table_h, idx_h, w_h, out_h, idx_v, w_v, rows_a, rows_b,
          out_a, out_b, sem_a, sem_b):
        wid = lax.axis_index("s") * info.num_cores + lax.axis_index("c")
        tok0 = wid * trows_pw
        pltpu.sync_copy(idx_h.at[pl.ds(tok0, trows_pw)], idx_v)
        pltpu.sync_copy(w_h.at[pl.ds(tok0, trows_pw)], w_v)

        def gather(g, rows_v, sem):
            for i in range(4):
                pltpu.async_copy(table_h.at[idx_v.at[4 * g + i]],
                                 rows_v.at[pl.ds(i * 32, 32)], sem)

        def gather_wait(g, rows_v, sem):
            for i in range(4):
                pltpu.make_async_copy(table_h.at[idx_v.at[4 * g + i]],
                                      rows_v.at[pl.ds(i * 32, 32)],
                                      sem).wait()

        def combine_store(g, rows_v, out_v):
            def per_tok(i, _):
                wlo = w_v.at[4 * g + i][pl.ds(0, 16)]
                whi = w_v.at[4 * g + i][pl.ds(16, 16)]
                for h in range(HEADS):
                    wsrc = wlo if h < 2 else whi
                    lane0 = (h % 2) * TOPK
                    ws = [wsrc[lane0 + kk] for kk in range(TOPK)]
                    for j in range(DH // 16):
                        acc = jnp.zeros((16,), jnp.float32)
                        for kk in range(TOPK):
                            acc = acc + ws[kk] * rows_v.at[
                                i * 32 + h * TOPK + kk][pl.ds(j * 16, 16)]
                        out_v.at[i][pl.ds(h * DH + j * 16, 16)] = acc
                return _

            lax.fori_loop(0, 4, per_tok, None)
            pltpu.sync_copy(out_v, out_h.at[pl.ds(tok0 + 4 * g, 4)])

        gather(0, rows_a, sem_a)

        def pair(i, _):
            g0 = 2 * i
            g1 = g0 + 1
            gather_wait(g0, rows_a, sem_a)
            gather(g1, rows_b, sem_b)
            combine_store(g0, rows_a, out_a)
            gather_wait(g1, rows_b, sem_b)

            @pl.when(i < ngrp // 2 - 1)
            def _start_next():
                gather(g1 + 1, rows_a, sem_a)

            combine_store(g1, rows_b, out_b)
            return _

        lax.fori_loop(0, ngrp // 2, pair, None)

    return k(table, idx, wts)


def _run_stage_a(tokens2d, keys, Wq, Wg, Wv, gq, gg, gv, blk0, nblk):
    n = nblk * BLK
    grid = nblk
    const = lambda shape: pl.BlockSpec(shape, lambda i: (0,) * len(shape))
    return pl.pallas_call(
        _stage_a,
        grid=(grid,),
        in_specs=[
            pl.BlockSpec((BLK, DIM), lambda i: (blk0 + i, 0)),
            const((HEADS, NUM_KEYS, DH)),
            const((HEADS, NUM_KEYS, DH)),
            const((DIM, DQK * 2)),
            const((1, DIM)),
            const((DIM, DV)),
            const((1, DIM)),
            const((1, DIM)),
            const((1, DIM)),
        ],
        out_specs=[
            pl.BlockSpec((BLK, HEADS * TOPK), lambda i: (i, 0)),
            pl.BlockSpec((BLK, HEADS * TOPK), lambda i: (i, 0)),
            pl.BlockSpec((BLK, DV), lambda i: (i, 0)),
            pl.BlockSpec((BLK, DH), lambda i: (i, 0)),
        ],
        out_shape=[
            jax.ShapeDtypeStruct((n, HEADS * TOPK), jnp.int32),
            jax.ShapeDtypeStruct((n, HEADS * TOPK), jnp.float32),
            jax.ShapeDtypeStruct((n, DV), jnp.float32),
            jax.ShapeDtypeStruct((n, DH), jnp.float32),
        ],
    )(tokens2d, keys[0], keys[1], Wq, Wg.reshape(1, DIM), Wv,
      gq.reshape(1, DIM), gg.reshape(1, DIM), gv.reshape(1, DIM))


def _run_stage_c(values, tvn, gate, go, Wo):
    n = tvn.shape[0]
    grid = n // BLK
    const = lambda shape: pl.BlockSpec(shape, lambda i: (0,) * len(shape))
    return pl.pallas_call(
        _stage_c,
        grid=(grid,),
        in_specs=[
            pl.BlockSpec((BLK, DV), lambda i: (i, 0)),
            pl.BlockSpec((BLK, DV), lambda i: (i, 0)),
            pl.BlockSpec((BLK, DH), lambda i: (i, 0)),
            const((1, DV)),
            const((DV, DIM)),
        ],
        out_specs=pl.BlockSpec((BLK, DIM), lambda i: (i, 0)),
        out_shape=jax.ShapeDtypeStruct((n, DIM), jnp.float32),
    )(values, tvn, gate, go.reshape(1, DV), Wo)


def kernel(tokens, memories, keys, Wq, Wg, Wv, Wo, gq, gg, gv, go):
    b, n, _ = tokens.shape
    tok2d = tokens.reshape(b * n, DIM)
    table = memories.reshape(-1, DH)
    nsplit = 2
    ntok = (b * n) // nsplit
    nblk = ntok // BLK
    parts = []
    for s in range(nsplit):
        fidx, wts, tvn, gate = _run_stage_a(tok2d, keys, Wq, Wg, Wv,
                                            gq, gg, gv, s * nblk, nblk)
        vals = _sc_gather_combine(table, fidx, wts)
        parts.append(_run_stage_c(vals, tvn, gate, go, Wo))
    return jnp.concatenate(parts, axis=0).reshape(b, n, DIM)


# speedup vs baseline: 22.1435x; 1.0475x over previous
import functools

import jax
import jax.numpy as jnp
from jax import lax
from jax.experimental import pallas as pl
from jax.experimental.pallas import tpu as pltpu
from jax.experimental.pallas import tpu_sc as plsc

DIM = 1024
HEADS = 4
NUM_KEYS = 256
DQK = 512
DV = 512
TOPK = 8
DH = 128
RMS_EPS = 1.1920929e-07
IDW_EPS = 0.001
BLK = 256
TCH = 128


def _rms(x, g):
    return x / jnp.sqrt(jnp.mean(x * x, axis=-1, keepdims=True) + RMS_EPS) * g


def _top8(d):
    iota = lax.broadcasted_iota(jnp.int32, d.shape, 0).astype(jnp.float32)
    work = d
    vals, poss = [], []
    for _ in range(TOPK):
        m = jnp.min(work, axis=0, keepdims=True)
        pos = jnp.min(jnp.where(work == m, iota, jnp.float32(512.0)),
                      axis=0, keepdims=True)
        work = jnp.where(iota == pos, jnp.float32(jnp.inf), work)
        vals.append(m)
        poss.append(pos)
    return (jnp.concatenate(poss, axis=0).astype(jnp.int32),
            jnp.concatenate(vals, axis=0))


def _sel8(arr, sel):
    out = jnp.zeros(sel.shape, arr.dtype)
    for i in range(TOPK):
        out = jnp.where(sel == i, arr[i:i + 1, :], out)
    return out


def _stage_a(tok, k1, k2, wq, wgr, wv, gq, gg, gv,
             fidx_o, wts_o, tvn_o, gate_o):
    x = tok[...]
    xq = _rms(x, gq[...])
    q = jnp.dot(xq, wq[...], preferred_element_type=jnp.float32)
    dn_t = (((1,), (1,)), ((), ()))
    for h in range(HEADS):
        k1h = k1[h]
        k2h = k2[h]
        ks1 = jnp.sum(k1h * k1h, axis=-1, keepdims=True)
        ks2 = jnp.sum(k2h * k2h, axis=-1, keepdims=True)
        for c in range(0, q.shape[0], TCH):
            sl = slice(c, c + TCH)
            q1 = q[sl, h * DH:(h + 1) * DH]
            q2 = q[sl, DQK + h * DH:DQK + (h + 1) * DH]
            c1 = lax.dot_general(k1h, q1, dn_t,
                                 preferred_element_type=jnp.float32)
            c2 = lax.dot_general(k2h, q2, dn_t,
                                 preferred_element_type=jnp.float32)
            qs1 = jnp.transpose(jnp.sum(q1 * q1, axis=-1, keepdims=True))
            qs2 = jnp.transpose(jnp.sum(q2 * q2, axis=-1, keepdims=True))
            idx1, s1 = _top8((qs1 + ks1) - 2.0 * c1 + IDW_EPS)
            idx2, s2 = _top8((qs2 + ks2) - 2.0 * c2 + IDW_EPS)
            prod = (s1[:, None, :] * s2[None, :, :]).reshape(
                TOPK * TOPK, -1)
            pos, pval = _top8(prod)
            g1 = _sel8(idx1, lax.shift_right_logical(pos, 3))
            g2 = _sel8(idx2, jnp.bitwise_and(pos, jnp.int32(7)))
            w = 1.0 / pval
            w = w / jnp.sum(w, axis=0, keepdims=True)
            fidx_o[sl, h * TOPK:(h + 1) * TOPK] = jnp.transpose(
                (g1 * NUM_KEYS + g2) * HEADS + h)
            wts_o[sl, h * TOPK:(h + 1) * TOPK] = jnp.transpose(w)

    xg = _rms(x, gg[...])
    gate = jax.nn.sigmoid(jnp.sum(xg * wgr[...], axis=-1, keepdims=True))
    xv = _rms(x, gv[...])
    tv = jnp.dot(xv, wv[...], preferred_element_type=jnp.float32)
    for h in range(HEADS):
        th = tv[:, h * DH:(h + 1) * DH]
        mu = jnp.mean(th, axis=-1, keepdims=True)
        ctr = th - mu
        std = jnp.sqrt(jnp.sum(ctr * ctr, axis=-1, keepdims=True) / (DH - 1))
        tvn_o[:, h * DH:(h + 1) * DH] = ctr / jnp.maximum(std, 1e-10)
    gate_o[...] = jnp.broadcast_to(gate, (gate.shape[0], DH))


def _stage_c(vals, tvn, gate, go, wo, out_o):
    g = gate[:, 0:1]
    o = tvn[...] + g * (vals[...] - tvn[...])
    on = _rms(o, go[...])
    out_o[...] = jnp.dot(on, wo[...], preferred_element_type=jnp.float32)


def _sc_gather_combine(table, idx, wts):
    info = plsc.get_sparse_core_info()
    nw = info.num_cores * info.num_subcores
    ntok = idx.shape[0]
    trows_pw = ntok // nw
    ngrp = trows_pw // 4
    mesh = plsc.VectorSubcoreMesh(core_axis_name="c", subcore_axis_name="s")

    @functools.partial(
        pl.kernel,
        out_type=jax.ShapeDtypeStruct((ntok, DV), jnp.float32),
        mesh=mesh,
        scratch_types=[
            pltpu.VMEM((trows_pw, 32), jnp.int32),
            pltpu.VMEM((trows_pw, 32), jnp.float32),
            pltpu.VMEM((128, DH), jnp.float32),
            pltpu.VMEM((128, DH), jnp.float32),
            pltpu.VMEM((4, DV), jnp.float32),
            pltpu.VMEM((4, DV), jnp.float32),
            pltpu.SemaphoreType.DMA,
            pltpu.SemaphoreType.DMA,
        ],
    )
    def k(table_h, idx_h, w_h, out_h, idx_v, w_v, rows_a, rows_b,
          out_a, out_b, sem_a, sem_b):
        wid = lax.axis_index("s") * info.num_cores + lax.axis_index("c")
        tok0 = wid * trows_pw
        pltpu.sync_copy(idx_h.at[pl.ds(tok0, trows_pw)], idx_v)
        pltpu.sync_copy(w_h.at[pl.ds(tok0, trows_pw)], w_v)

        def gather(g, rows_v, sem):
            for i in range(4):
                pltpu.async_copy(table_h.at[idx_v.at[4 * g + i]],
                                 rows_v.at[pl.ds(i * 32, 32)], sem)

        def gather_wait(g, rows_v, sem):
            for i in range(4):
                pltpu.make_async_copy(table_h.at[idx_v.at[4 * g + i]],
                                      rows_v.at[pl.ds(i * 32, 32)],
                                      sem).wait()

        def combine_store(g, rows_v, out_v):
            def per_tok(i, _):
                wlo = w_v.at[4 * g + i][pl.ds(0, 16)]
                whi = w_v.at[4 * g + i][pl.ds(16, 16)]
                for h in range(HEADS):
                    wsrc = wlo if h < 2 else whi
                    lane0 = (h % 2) * TOPK
                    ws = [wsrc[lane0 + kk] for kk in range(TOPK)]
                    for j in range(DH // 16):
                        acc = jnp.zeros((16,), jnp.float32)
                        for kk in range(TOPK):
                            acc = acc + ws[kk] * rows_v.at[
                                i * 32 + h * TOPK + kk][pl.ds(j * 16, 16)]
                        out_v.at[i][pl.ds(h * DH + j * 16, 16)] = acc
                return _

            lax.fori_loop(0, 4, per_tok, None)
            pltpu.sync_copy(out_v, out_h.at[pl.ds(tok0 + 4 * g, 4)])

        gather(0, rows_a, sem_a)

        def pair(i, _):
            g0 = 2 * i
            g1 = g0 + 1
            gather_wait(g0, rows_a, sem_a)
            gather(g1, rows_b, sem_b)
            combine_store(g0, rows_a, out_a)
            gather_wait(g1, rows_b, sem_b)

            @pl.when(i < ngrp // 2 - 1)
            def _start_next():
                gather(g1 + 1, rows_a, sem_a)

            combine_store(g1, rows_b, out_b)
            return _

        lax.fori_loop(0, ngrp // 2, pair, None)

    return k(table, idx, wts)


def _run_stage_a(tokens2d, keys, Wq, Wg, Wv, gq, gg, gv, blk0, nblk):
    n = nblk * BLK
    grid = nblk
    const = lambda shape: pl.BlockSpec(shape, lambda i: (0,) * len(shape))
    return pl.pallas_call(
        _stage_a,
        grid=(grid,),
        in_specs=[
            pl.BlockSpec((BLK, DIM), lambda i: (blk0 + i, 0)),
            const((HEADS, NUM_KEYS, DH)),
            const((HEADS, NUM_KEYS, DH)),
            const((DIM, DQK * 2)),
            const((1, DIM)),
            const((DIM, DV)),
            const((1, DIM)),
            const((1, DIM)),
            const((1, DIM)),
        ],
        out_specs=[
            pl.BlockSpec((BLK, HEADS * TOPK), lambda i: (i, 0)),
            pl.BlockSpec((BLK, HEADS * TOPK), lambda i: (i, 0)),
            pl.BlockSpec((BLK, DV), lambda i: (i, 0)),
            pl.BlockSpec((BLK, DH), lambda i: (i, 0)),
        ],
        out_shape=[
            jax.ShapeDtypeStruct((n, HEADS * TOPK), jnp.int32),
            jax.ShapeDtypeStruct((n, HEADS * TOPK), jnp.float32),
            jax.ShapeDtypeStruct((n, DV), jnp.float32),
            jax.ShapeDtypeStruct((n, DH), jnp.float32),
        ],
    )(tokens2d, keys[0], keys[1], Wq, Wg.reshape(1, DIM), Wv,
      gq.reshape(1, DIM), gg.reshape(1, DIM), gv.reshape(1, DIM))


def _run_stage_c(values, tvn, gate, go, Wo):
    n = tvn.shape[0]
    grid = n // BLK
    const = lambda shape: pl.BlockSpec(shape, lambda i: (0,) * len(shape))
    return pl.pallas_call(
        _stage_c,
        grid=(grid,),
        in_specs=[
            pl.BlockSpec((BLK, DV), lambda i: (i, 0)),
            pl.BlockSpec((BLK, DV), lambda i: (i, 0)),
            pl.BlockSpec((BLK, DH), lambda i: (i, 0)),
            const((1, DV)),
            const((DV, DIM)),
        ],
        out_specs=pl.BlockSpec((BLK, DIM), lambda i: (i, 0)),
        out_shape=jax.ShapeDtypeStruct((n, DIM), jnp.float32),
    )(values, tvn, gate, go.reshape(1, DV), Wo)


def kernel(tokens, memories, keys, Wq, Wg, Wv, Wo, gq, gg, gv, go):
    b, n, _ = tokens.shape
    tok2d = tokens.reshape(b * n, DIM)
    table = memories.reshape(-1, DH)
    nsplit = 4
    ntok = (b * n) // nsplit
    nblk = ntok // BLK
    parts = []
    for s in range(nsplit):
        fidx, wts, tvn, gate = _run_stage_a(tok2d, keys, Wq, Wg, Wv,
                                            gq, gg, gv, s * nblk, nblk)
        vals = _sc_gather_combine(table, fidx, wts)
        parts.append(_run_stage_c(vals, tvn, gate, go, Wo))
    return jnp.concatenate(parts, axis=0).reshape(b, n, DIM)
